# baseline (speedup 1.0000x reference)
import functools

import jax
import jax.numpy as jnp
from jax import lax
from jax.experimental import pallas as pl
from jax.experimental.pallas import tpu as pltpu
from jax.experimental.pallas import tpu_sc as plsc

L = 64
N = 8192
V = 128
LANES = 16
NC, NS = 2, 16
NW = NC * NS
RW = L // NW
VECS = N // LANES


def _body(x_hbm, vals_hbm, q_hbm, i_hbm, x_v, vals_v, q_v, i_v):
    wid = lax.axis_index("s") * NC + lax.axis_index("c")
    row0 = wid * RW
    pltpu.sync_copy(x_hbm.at[pl.ds(row0, RW)], x_v)
    for r in range(RW):
        pltpu.sync_copy(vals_hbm.at[row0 + r], vals_v.at[pl.ds(r * V, V)])

    for r in range(RW):
        base = jnp.full((LANES,), r * V, jnp.int32)
        head = vals_v[pl.ds(r * V, LANES)]
        tail = vals_v[pl.ds(r * V + V - LANES, LANES)]
        v0 = jnp.full((LANES,), head[0])
        vL = jnp.full((LANES,), tail[LANES - 1])
        scale = jnp.float32(V - 1) / (vL - v0)

        @plsc.parallel_loop(0, N, LANES, unroll=8)
        def step(off):
            xv = x_v[r, pl.ds(off, LANES)]
            t = (xv - v0) * scale
            t = jnp.minimum(jnp.maximum(t, jnp.float32(0.0)), jnp.float32(V - 1))
            k0 = (t + jnp.float32(0.5)).astype(jnp.int32)
            a = jnp.maximum(k0 - 1, 0)
            c = jnp.minimum(k0 + 1, V - 1)
            va = plsc.load_gather(vals_v, [base + a])
            vb = plsc.load_gather(vals_v, [base + k0])
            vc = plsc.load_gather(vals_v, [base + c])
            da = jnp.abs(xv - va)
            db = jnp.abs(xv - vb)
            dc = jnp.abs(xv - vc)
            k = a + (db < da).astype(jnp.int32) + (dc < jnp.minimum(da, db)).astype(jnp.int32)
            q = plsc.load_gather(vals_v, [base + k])
            q_v[r, pl.ds(off, LANES)] = q
            i_v[r, pl.ds(off, LANES)] = k

    pltpu.sync_copy(q_v, q_hbm.at[pl.ds(row0, RW)])
    pltpu.sync_copy(i_v, i_hbm.at[pl.ds(row0, RW)])


_quantize = functools.partial(
    pl.kernel,
    mesh=plsc.VectorSubcoreMesh(core_axis_name="c", subcore_axis_name="s"),
    out_type=[
        jax.ShapeDtypeStruct((L, N), jnp.float32),
        jax.ShapeDtypeStruct((L, N), jnp.int32),
    ],
    scratch_types=[
        pltpu.VMEM((RW, N), jnp.float32),
        pltpu.VMEM((RW * V,), jnp.float32),
        pltpu.VMEM((RW, N), jnp.float32),
        pltpu.VMEM((RW, N), jnp.int32),
    ],
    compiler_params=pltpu.CompilerParams(needs_layout_passes=False),
)(_body)


def kernel(x, values):
    q, i = _quantize(x, values)
    return q, i


# speedup vs baseline: 1.0136x; 1.0136x over previous
import functools

import jax
import jax.numpy as jnp
from jax import lax
from jax.experimental import pallas as pl
from jax.experimental.pallas import tpu as pltpu
from jax.experimental.pallas import tpu_sc as plsc

L = 64
N = 8192
V = 128
LANES = 16
NC, NS = 2, 16
NW = NC * NS
RW = L // NW
VECS = N // LANES


def _body(x_hbm, vals_hbm, q_hbm, i_hbm, x_v, vals_v, q_v, i_v, sem_in, sem_out):
    wid = lax.axis_index("s") * NC + lax.axis_index("c")
    row0 = wid * RW
    in_copies = [
        pltpu.async_copy(x_hbm.at[row0 + r], x_v.at[r], sem_in.at[r])
        for r in range(RW)
    ]
    for r in range(RW):
        pltpu.sync_copy(vals_hbm.at[row0 + r], vals_v.at[pl.ds(r * V, V)])

    out_copies = []
    for r in range(RW):
        in_copies[r].wait()
        base = jnp.full((LANES,), r * V, jnp.int32)
        head = vals_v[pl.ds(r * V, LANES)]
        tail = vals_v[pl.ds(r * V + V - LANES, LANES)]
        v0 = jnp.full((LANES,), head[0])
        vL = jnp.full((LANES,), tail[LANES - 1])
        scale = jnp.float32(V - 1) / (vL - v0)

        @plsc.parallel_loop(0, N, LANES, unroll=8)
        def step(off):
            xv = x_v[r, pl.ds(off, LANES)]
            t = (xv - v0) * scale
            t = jnp.minimum(jnp.maximum(t, jnp.float32(0.0)), jnp.float32(V - 1))
            k0 = (t + jnp.float32(0.5)).astype(jnp.int32)
            a = jnp.maximum(k0 - 1, 0)
            c = jnp.minimum(k0 + 1, V - 1)
            va = plsc.load_gather(vals_v, [base + a])
            vb = plsc.load_gather(vals_v, [base + k0])
            vc = plsc.load_gather(vals_v, [base + c])
            da = jnp.abs(xv - va)
            db = jnp.abs(xv - vb)
            dc = jnp.abs(xv - vc)
            k = a + (db < da).astype(jnp.int32) + (dc < jnp.minimum(da, db)).astype(jnp.int32)
            q = plsc.load_gather(vals_v, [base + k])
            q_v[r, pl.ds(off, LANES)] = q
            i_v[r, pl.ds(off, LANES)] = k

        out_copies.append(
            pltpu.async_copy(q_v.at[r], q_hbm.at[row0 + r], sem_out))
        out_copies.append(
            pltpu.async_copy(i_v.at[r], i_hbm.at[row0 + r], sem_out))

    for cp in out_copies:
        cp.wait()


_quantize = functools.partial(
    pl.kernel,
    mesh=plsc.VectorSubcoreMesh(core_axis_name="c", subcore_axis_name="s"),
    out_type=[
        jax.ShapeDtypeStruct((L, N), jnp.float32),
        jax.ShapeDtypeStruct((L, N), jnp.int32),
    ],
    scratch_types=[
        pltpu.VMEM((RW, N), jnp.float32),
        pltpu.VMEM((RW * V,), jnp.float32),
        pltpu.VMEM((RW, N), jnp.float32),
        pltpu.VMEM((RW, N), jnp.int32),
        pltpu.SemaphoreType.DMA((RW,)),
        pltpu.SemaphoreType.DMA,
    ],
    compiler_params=pltpu.CompilerParams(needs_layout_passes=False),
)(_body)


def kernel(x, values):
    q, i = _quantize(x, values)
    return q, i
